# Initial kernel scaffold; baseline (speedup 1.0000x reference)
import jax
import jax.numpy as jnp
from jax.experimental import pallas as pl
from jax.experimental.pallas import tpu as pltpu

E = 8
K = 2
D = 768
H = 3072
N = 256
NP = N * K
BR = 64
NBLK = 16
PADTOT = NBLK * BR
HC = 768
HCN = H // HC


def _route_body(inp_ref, wg_ref, bg_ref,
                xs_ref, pos_ref, scl_ref, eid_ref, nlive_ref):
    x = inp_ref[...]
    wg = wg_ref[...]
    logits = jax.lax.dot_general(
        x, wg, (((1,), (1,)), ((), ())), preferred_element_type=jnp.float32)
    logits = logits + bg_ref[...]

    ie = jax.lax.broadcasted_iota(jnp.int32, (N, E), 1)
    v1 = jnp.max(logits, axis=1, keepdims=True)
    i1 = jnp.min(jnp.where(logits == v1, ie, E), axis=1, keepdims=True)
    masked = jnp.where(ie == i1, -jnp.inf, logits)
    v2 = jnp.max(masked, axis=1, keepdims=True)
    i2 = jnp.min(jnp.where(masked == v2, ie, E), axis=1, keepdims=True)
    eb = jnp.exp(v2 - v1)
    s1 = 1.0 / (1.0 + eb)
    s2 = eb / (1.0 + eb)

    oh0 = (i1 == ie).astype(jnp.float32)
    oh1 = (i2 == ie).astype(jnp.float32)
    ohsum = oh0 + oh1
    c = jnp.sum(ohsum, axis=0, keepdims=True)
    rt = jax.lax.broadcasted_iota(jnp.int32, (N, N), 0)
    ct = jax.lax.broadcasted_iota(jnp.int32, (N, N), 1)
    lt = (ct < rt).astype(jnp.float32)
    cum = jax.lax.dot_general(
        lt, ohsum, (((1,), (0,)), ((), ())), preferred_element_type=jnp.float32)

    cb = jnp.floor((c + (BR - 1.0)) / BR)
    e8 = jax.lax.broadcasted_iota(jnp.float32, (E, E), 0)
    f8 = jax.lax.broadcasted_iota(jnp.float32, (E, E), 1)
    excl = (e8 < f8).astype(jnp.float32)
    blk_off = jax.lax.dot_general(
        cb, excl, (((1,), (0,)), ((), ())), preferred_element_type=jnp.float32)
    pad_off = blk_off * BR

    pos0 = jnp.sum(oh0 * (pad_off + cum), axis=1, keepdims=True)
    pos1 = jnp.sum(oh1 * (pad_off + cum), axis=1, keepdims=True)

    ip = jax.lax.broadcasted_iota(jnp.float32, (N, PADTOT), 1)
    ohp0 = (ip == pos0).astype(jnp.float32)
    ohp1 = (ip == pos1).astype(jnp.float32)
    xs = jax.lax.dot_general(
        ohp0, x, (((0,), (0,)), ((), ())), preferred_element_type=jnp.float32)
    xs = xs + jax.lax.dot_general(
        ohp1, x, (((0,), (0,)), ((), ())), preferred_element_type=jnp.float32)
    xs_ref[...] = xs

    pos_ref[...] = jnp.concatenate([pos0, pos1], axis=1)
    scl_ref[...] = jnp.concatenate([s1, s2], axis=1)

    b16 = jax.lax.broadcasted_iota(jnp.float32, (NBLK, E), 0)
    eid = jnp.sum((blk_off <= b16).astype(jnp.float32), axis=1) - 1.0
    eid_ref[...] = eid.astype(jnp.int32)[None, :]
    nlive_ref[...] = jnp.sum(cb, axis=1, keepdims=True).astype(jnp.int32)


def _route(inp, Wg, bg):
    return pl.pallas_call(
        _route_body,
        out_shape=(
            jax.ShapeDtypeStruct((PADTOT, D), jnp.float32),
            jax.ShapeDtypeStruct((N, K), jnp.float32),
            jax.ShapeDtypeStruct((N, K), jnp.float32),
            jax.ShapeDtypeStruct((1, NBLK), jnp.int32),
            jax.ShapeDtypeStruct((1, 1), jnp.int32),
        ),
        interpret=False,
    )(inp, Wg, bg)


def _ffn_body(nl_ref, eid_ref, xs_ref, w1_ref, b1_ref, w2_ref, b2_ref, y_ref):
    b = pl.program_id(0)
    hc = pl.program_id(1)
    live = b < nl_ref[0]

    @pl.when(live)
    def _():
        xb = xs_ref[...]
        w1 = w1_ref[0]
        h = jax.lax.dot_general(
            xb, w1, (((1,), (1,)), ((), ())), preferred_element_type=jnp.float32)
        h = h + b1_ref[0]
        h = jax.nn.gelu(h, approximate=False)
        w2 = w2_ref[0]
        y = jax.lax.dot_general(
            h, w2, (((1,), (1,)), ((), ())), preferred_element_type=jnp.float32)

        @pl.when(hc == 0)
        def _():
            y_ref[...] = y + b2_ref[0]

        @pl.when(hc > 0)
        def _():
            y_ref[...] = y_ref[...] + y

    @pl.when(jnp.logical_and(jnp.logical_not(live), hc == 0))
    def _():
        y_ref[...] = jnp.zeros_like(y_ref)


def _ffn(nlive, eid, xs, W1, b1, W2, b2):
    def xmap(b, hc, nl, eid):
        return (jnp.minimum(b, nl[0] - 1), 0)

    def w1map(b, hc, nl, eid):
        bl = jnp.minimum(b, nl[0] - 1)
        return (eid[bl], jnp.where(b < nl[0], hc, HCN - 1), 0)

    def b1map(b, hc, nl, eid):
        bl = jnp.minimum(b, nl[0] - 1)
        return (eid[bl], 0, jnp.where(b < nl[0], hc, HCN - 1))

    def w2map(b, hc, nl, eid):
        bl = jnp.minimum(b, nl[0] - 1)
        return (eid[bl], 0, jnp.where(b < nl[0], hc, HCN - 1))

    def b2map(b, hc, nl, eid):
        return (eid[jnp.minimum(b, nl[0] - 1)], 0, 0)

    def ymap(b, hc, nl, eid):
        return (b, 0)

    grid_spec = pltpu.PrefetchScalarGridSpec(
        num_scalar_prefetch=2,
        grid=(NBLK, HCN),
        in_specs=[
            pl.BlockSpec((BR, D), xmap),
            pl.BlockSpec((1, HC, D), w1map),
            pl.BlockSpec((1, 1, HC), b1map),
            pl.BlockSpec((1, D, HC), w2map),
            pl.BlockSpec((1, 1, D), b2map),
        ],
        out_specs=pl.BlockSpec((BR, D), ymap),
    )
    return pl.pallas_call(
        _ffn_body,
        grid_spec=grid_spec,
        out_shape=jax.ShapeDtypeStruct((PADTOT, D), jnp.float32),
        compiler_params=pltpu.CompilerParams(
            dimension_semantics=("arbitrary", "arbitrary")),
        interpret=False,
    )(nlive, eid, xs, W1, b1, W2, b2)


def _combine_body(pos_ref, scl_ref, y_ref, g_ref, bt_ref, out_ref):
    pos = pos_ref[...]
    scl = scl_ref[...]
    ip = jax.lax.broadcasted_iota(jnp.float32, (N, PADTOT), 1)
    u = jnp.where(ip == pos[:, 0:1], scl[:, 0:1], 0.0)
    u = u + jnp.where(ip == pos[:, 1:2], scl[:, 1:2], 0.0)
    o = jax.lax.dot_general(
        u, y_ref[...], (((1,), (0,)), ((), ())), preferred_element_type=jnp.float32)
    mu = jnp.mean(o, axis=1, keepdims=True)
    var = jnp.mean((o - mu) ** 2, axis=1, keepdims=True)
    o = (o - mu) / jnp.sqrt(var + 1e-5) * g_ref[...] + bt_ref[...]
    out_ref[...] = o


def _combine(pos, scl, y, gamma, beta):
    return pl.pallas_call(
        _combine_body,
        out_shape=jax.ShapeDtypeStruct((N, D), jnp.float32),
        interpret=False,
    )(pos, scl, y, gamma, beta)


def kernel(inp, Wg, bg, W1, b1, W2, b2, gamma, beta):
    xs, pos, scl, eid, nlive = _route(inp, Wg, bg.reshape(1, E))
    y = _ffn(nlive.reshape(1), eid.reshape(NBLK),
             xs, W1, b1.reshape(E, 1, H), W2, b2.reshape(E, 1, D))
    out = _combine(pos, scl, y, gamma.reshape(1, D), beta.reshape(1, D))
    return out


# speedup vs baseline: 2.2312x; 2.2312x over previous
import jax
import jax.numpy as jnp
from jax.experimental import pallas as pl
from jax.experimental.pallas import tpu as pltpu

E = 8
K = 2
D = 768
H = 3072
N = 256
NP = N * K
BR = 64
NBLK = 16
PADTOT = NBLK * BR


def _route_body(inp_ref, wg_ref, bg_ref, pos_ref, scl_ref, eid_ref, nlive_ref):
    x = inp_ref[...]
    wg = wg_ref[...]
    logits = jax.lax.dot_general(
        x, wg, (((1,), (1,)), ((), ())), preferred_element_type=jnp.float32)
    logits = logits + bg_ref[...]

    ie = jax.lax.broadcasted_iota(jnp.int32, (N, E), 1)
    v1 = jnp.max(logits, axis=1, keepdims=True)
    i1 = jnp.min(jnp.where(logits == v1, ie, E), axis=1, keepdims=True)
    masked = jnp.where(ie == i1, -jnp.inf, logits)
    v2 = jnp.max(masked, axis=1, keepdims=True)
    i2 = jnp.min(jnp.where(masked == v2, ie, E), axis=1, keepdims=True)
    eb = jnp.exp(v2 - v1)
    s1 = 1.0 / (1.0 + eb)
    s2 = eb / (1.0 + eb)

    oh0 = (i1 == ie).astype(jnp.float32)
    oh1 = (i2 == ie).astype(jnp.float32)
    ohsum = oh0 + oh1
    c = jnp.sum(ohsum, axis=0, keepdims=True)
    rt = jax.lax.broadcasted_iota(jnp.int32, (N, N), 0)
    ct = jax.lax.broadcasted_iota(jnp.int32, (N, N), 1)
    lt = (ct < rt).astype(jnp.float32)
    cum = jax.lax.dot_general(
        lt, ohsum, (((1,), (0,)), ((), ())), preferred_element_type=jnp.float32)

    cb = jnp.floor((c + (BR - 1.0)) / BR)
    e8 = jax.lax.broadcasted_iota(jnp.int32, (E, E), 0)
    f8 = jax.lax.broadcasted_iota(jnp.int32, (E, E), 1)
    excl = (e8 < f8).astype(jnp.float32)
    blk_off = jax.lax.dot_general(
        cb, excl, (((1,), (0,)), ((), ())), preferred_element_type=jnp.float32)
    pad_off = blk_off * BR

    pos0 = jnp.sum(oh0 * (pad_off + cum), axis=1, keepdims=True)
    pos1 = jnp.sum(oh1 * (pad_off + cum), axis=1, keepdims=True)

    pos_ref[...] = jnp.concatenate([pos0, pos1], axis=1)
    scl_ref[...] = jnp.concatenate([s1, s2], axis=1)

    b16 = jax.lax.broadcasted_iota(jnp.int32, (NBLK, E), 0).astype(jnp.float32)
    eid = jnp.sum((blk_off <= b16).astype(jnp.float32), axis=1) - 1.0
    eid_ref[...] = eid.astype(jnp.int32)[None, :]
    nlive_ref[...] = jnp.sum(cb, axis=1, keepdims=True).astype(jnp.int32)


def _route(inp, Wg, bg):
    return pl.pallas_call(
        _route_body,
        out_shape=(
            jax.ShapeDtypeStruct((N, K), jnp.float32),
            jax.ShapeDtypeStruct((N, K), jnp.float32),
            jax.ShapeDtypeStruct((1, NBLK), jnp.int32),
            jax.ShapeDtypeStruct((1, 1), jnp.int32),
        ),
        interpret=False,
    )(inp, Wg, bg)


def _ffn_body(nl_ref, eid_ref, inp_ref, pos_ref, scl_ref,
              w1_ref, b1_ref, w2_ref, b2_ref, g_ref, bt_ref, out_ref, acc_ref):
    b = pl.program_id(0)
    live = b < nl_ref[0]

    @pl.when(live)
    def _():
        posi = pos_ref[...].astype(jnp.int32)
        scl = scl_ref[...]
        slot = jax.lax.broadcasted_iota(jnp.int32, (N, BR), 1) + b * BR
        g0 = posi[:, 0:1] == slot
        g1 = posi[:, 1:2] == slot
        gb = jnp.logical_or(g0, g1).astype(jnp.float32)
        xb = jax.lax.dot_general(
            gb, inp_ref[...], (((0,), (0,)), ((), ())),
            preferred_element_type=jnp.float32)
        w1 = w1_ref[0]
        h = jax.lax.dot_general(
            xb, w1, (((1,), (1,)), ((), ())), preferred_element_type=jnp.float32)
        h = h + b1_ref[0]
        h = 0.5 * h * (1.0 + jax.lax.erf(h * 0.7071067811865476))
        w2 = w2_ref[0]
        y = jax.lax.dot_general(
            h, w2, (((1,), (1,)), ((), ())), preferred_element_type=jnp.float32)
        y = y + b2_ref[0]
        ub = (jnp.where(g0, scl[:, 0:1], 0.0)
              + jnp.where(g1, scl[:, 1:2], 0.0))
        contrib = jax.lax.dot_general(
            ub, y, (((1,), (0,)), ((), ())), preferred_element_type=jnp.float32)

        @pl.when(b == 0)
        def _():
            acc_ref[...] = contrib

        @pl.when(b > 0)
        def _():
            acc_ref[...] = acc_ref[...] + contrib

    @pl.when(b == NBLK)
    def _():
        o = acc_ref[...]
        mu = jnp.mean(o, axis=1, keepdims=True)
        var = jnp.mean((o - mu) ** 2, axis=1, keepdims=True)
        out_ref[...] = ((o - mu) / jnp.sqrt(var + 1e-5)) * g_ref[...] + bt_ref[...]


def _ffn(nlive, eid, inp, pos, scl, W1, b1, W2, b2, gamma, beta):
    def cmap(b, nl, eid):
        return (0, 0)

    def emap(b, nl, eid):
        return (eid[jnp.minimum(b, nl[0] - 1)], 0, 0)

    grid_spec = pltpu.PrefetchScalarGridSpec(
        num_scalar_prefetch=2,
        grid=(NBLK + 1,),
        in_specs=[
            pl.BlockSpec((N, D), cmap),
            pl.BlockSpec((N, K), cmap),
            pl.BlockSpec((N, K), cmap),
            pl.BlockSpec((1, H, D), emap),
            pl.BlockSpec((1, 1, H), emap),
            pl.BlockSpec((1, D, H), emap),
            pl.BlockSpec((1, 1, D), emap),
            pl.BlockSpec((1, D), cmap),
            pl.BlockSpec((1, D), cmap),
        ],
        out_specs=pl.BlockSpec((N, D), cmap),
        scratch_shapes=[pltpu.VMEM((N, D), jnp.float32)],
    )
    return pl.pallas_call(
        _ffn_body,
        grid_spec=grid_spec,
        out_shape=jax.ShapeDtypeStruct((N, D), jnp.float32),
        compiler_params=pltpu.CompilerParams(
            dimension_semantics=("arbitrary",)),
        interpret=False,
    )(nlive, eid, inp, pos, scl, W1, b1, W2, b2, gamma, beta)


def kernel(inp, Wg, bg, W1, b1, W2, b2, gamma, beta):
    pos, scl, eid, nlive = _route(inp, Wg, bg.reshape(1, E))
    return _ffn(nlive.reshape(1), eid.reshape(NBLK), inp, pos, scl,
                W1, b1.reshape(E, 1, H), W2, b2.reshape(E, 1, D),
                gamma.reshape(1, D), beta.reshape(1, D))
